# baseline (speedup 1.0000x reference)
import functools
import jax
import jax.numpy as jnp
from jax import lax
from jax.experimental import pallas as pl
from jax.experimental.pallas import tpu as pltpu
from jax.experimental.pallas import tpu_sc as plsc

_B = 16384
_C = 26
_D = 64
_NC = 2
_NS = 16
_NW = _NC * _NS
_S_SC = 4096
_BPW = _S_SC // _NW
_DCH = 8
_LANES = 16
_BB = 512


def _select_kernel(xt_hbm, label_hbm, out_hbm, lab_v, buf0, buf1, stage_v,
                   sem0, sem1):
    wid = lax.axis_index("s") * _NC + lax.axis_index("c")
    base = wid * _BPW

    pltpu.sync_copy(label_hbm.at[pl.ds(base, _BPW)], lab_v)

    bufs = (buf0, buf1)
    sems = (sem0, sem1)
    lane = lax.iota(jnp.int32, _LANES)

    n_dc = _D // _DCH

    def fire(k):
        pltpu.async_copy(
            xt_hbm.at[:, pl.ds(k * _DCH, _DCH), pl.ds(base, _BPW)],
            bufs[k % 2],
            sems[k % 2],
        )

    fire(0)
    fire(1)

    for dc in range(n_dc):
        buf, sem = bufs[dc % 2], sems[dc % 2]
        pltpu.make_async_copy(
            xt_hbm.at[:, pl.ds(0, _DCH), pl.ds(0, _BPW)], buf, sem
        ).wait()

        for bs in range(_BPW // _LANES):
            labs = lab_v[pl.ds(bs * _LANES, _LANES)]
            bidx = lane + bs * _LANES
            for d in range(_DCH):
                didx = jnp.full((_LANES,), d, dtype=jnp.int32)
                val = plsc.load_gather(buf, [labs, didx, bidx])
                stage_v[dc * _DCH + d, pl.ds(bs * _LANES, _LANES)] = val

        if dc + 2 < n_dc:
            fire(dc + 2)

    pltpu.sync_copy(stage_v, out_hbm.at[:, pl.ds(base, _BPW)])


@jax.jit
def _maskout(xt, label):
    mesh = plsc.VectorSubcoreMesh(core_axis_name="c", subcore_axis_name="s")
    out_sc = pl.kernel(
        _select_kernel,
        mesh=mesh,
        out_type=jax.ShapeDtypeStruct((_D, _B), jnp.float32),
        scratch_types=[
            pltpu.VMEM((_BPW,), jnp.int32),
            pltpu.VMEM((_C, _DCH, _BPW), jnp.float32),
            pltpu.VMEM((_C, _DCH, _BPW), jnp.float32),
            pltpu.VMEM((_D, _BPW), jnp.float32),
            pltpu.SemaphoreType.DMA,
            pltpu.SemaphoreType.DMA,
        ],
        compiler_params=pltpu.CompilerParams(
            use_tc_tiling_on_sc=True, needs_layout_passes=False
        ),
    )(xt, label)

    n_tc = _B - _S_SC
    lab3d = label.reshape(_B // _BB, 1, _BB)

    def _tc_body(x_ref, lab_ref, sc_ref, o_ref):
        del sc_ref
        labb = lab_ref[0]
        acc = x_ref[0]
        for c in range(1, _C):
            acc = jnp.where(labb == c, x_ref[c], acc)
        o_ref[...] = acc

    return pl.pallas_call(
        _tc_body,
        grid=(n_tc // _BB,),
        in_specs=[
            pl.BlockSpec((_C, _D, _BB), lambda i: (0, 0, i + _S_SC // _BB)),
            pl.BlockSpec((1, 1, _BB), lambda i: (i + _S_SC // _BB, 0, 0)),
            pl.BlockSpec(memory_space=pltpu.MemorySpace.HBM),
        ],
        out_specs=pl.BlockSpec((_D, _BB), lambda i: (0, i + _S_SC // _BB)),
        out_shape=jax.ShapeDtypeStruct((_D, _B), jnp.float32),
        input_output_aliases={2: 0},
    )(xt, lab3d, out_sc)


def kernel(x, label):
    xt = jnp.transpose(x, (1, 2, 0))
    out_t = _maskout(xt, label.astype(jnp.int32))
    return jnp.transpose(out_t, (1, 0))


# speedup vs baseline: 1.1567x; 1.1567x over previous
import functools
import jax
import jax.numpy as jnp
from jax import lax
from jax.experimental import pallas as pl
from jax.experimental.pallas import tpu as pltpu
from jax.experimental.pallas import tpu_sc as plsc

_B = 16384
_C = 26
_D = 64
_NC = 2
_NS = 16
_NW = _NC * _NS
_S_SC = 4096
_BPW = _S_SC // _NW
_DCH = 8
_LANES = 16
_BB = 512


def _select_kernel(xt_hbm, label_hbm, out_hbm, lab_v, buf0, buf1, stage_v,
                   sem0, sem1):
    wid = lax.axis_index("s") * _NC + lax.axis_index("c")
    base = wid * _BPW

    pltpu.sync_copy(label_hbm.at[pl.ds(base, _BPW)], lab_v)

    bufs = (buf0, buf1)
    sems = (sem0, sem1)
    lane = lax.iota(jnp.int32, _LANES)

    n_dc = _D // _DCH

    def fire(k):
        pltpu.async_copy(
            xt_hbm.at[:, pl.ds(k * _DCH, _DCH), pl.ds(base, _BPW)],
            bufs[k % 2],
            sems[k % 2],
        )

    fire(0)
    fire(1)

    for dc in range(n_dc):
        buf, sem = bufs[dc % 2], sems[dc % 2]
        pltpu.make_async_copy(
            xt_hbm.at[:, pl.ds(0, _DCH), pl.ds(0, _BPW)], buf, sem
        ).wait()

        for bs in range(_BPW // _LANES):
            labs = lab_v[pl.ds(bs * _LANES, _LANES)]
            bidx = lane + bs * _LANES
            for d in range(_DCH):
                didx = jnp.full((_LANES,), d, dtype=jnp.int32)
                val = plsc.load_gather(buf, [labs, didx, bidx])
                stage_v[dc * _DCH + d, pl.ds(bs * _LANES, _LANES)] = val

        if dc + 2 < n_dc:
            fire(dc + 2)

    pltpu.sync_copy(stage_v, out_hbm.at[:, pl.ds(base, _BPW)])


@jax.jit
def _maskout(xt, label):
    mesh = plsc.VectorSubcoreMesh(core_axis_name="c", subcore_axis_name="s")
    out_sc = pl.kernel(
        _select_kernel,
        mesh=mesh,
        out_type=jax.ShapeDtypeStruct((_D, _S_SC), jnp.float32),
        scratch_types=[
            pltpu.VMEM((_BPW,), jnp.int32),
            pltpu.VMEM((_C, _DCH, _BPW), jnp.float32),
            pltpu.VMEM((_C, _DCH, _BPW), jnp.float32),
            pltpu.VMEM((_D, _BPW), jnp.float32),
            pltpu.SemaphoreType.DMA,
            pltpu.SemaphoreType.DMA,
        ],
        compiler_params=pltpu.CompilerParams(
            use_tc_tiling_on_sc=True, needs_layout_passes=False
        ),
    )(xt, label)

    n_tc = _B - _S_SC
    lab3d = label.reshape(_B // _BB, 1, _BB)

    def _tc_body(x_ref, lab_ref, o_ref):
        labb = lab_ref[0]
        acc = x_ref[0]
        for c in range(1, _C):
            acc = jnp.where(labb == c, x_ref[c], acc)
        o_ref[...] = acc

    out_tc = pl.pallas_call(
        _tc_body,
        grid=(n_tc // _BB,),
        in_specs=[
            pl.BlockSpec((_C, _D, _BB), lambda i: (0, 0, i + _S_SC // _BB)),
            pl.BlockSpec((1, 1, _BB), lambda i: (i + _S_SC // _BB, 0, 0)),
        ],
        out_specs=pl.BlockSpec((_D, _BB), lambda i: (0, i)),
        out_shape=jax.ShapeDtypeStruct((_D, n_tc), jnp.float32),
    )(xt, lab3d)

    return jnp.concatenate([out_sc, out_tc], axis=1)


def kernel(x, label):
    xt = jnp.transpose(x, (1, 2, 0))
    out_t = _maskout(xt, label.astype(jnp.int32))
    return jnp.transpose(out_t, (1, 0))
